# baseline (speedup 1.0000x reference)
import functools

import jax
import jax.numpy as jnp
from jax import lax
from jax.experimental import pallas as pl
from jax.experimental.pallas import tpu as pltpu
from jax.experimental.pallas import tpu_sc as plsc

E = 16384
NG = 16
MARGIN = 0.1
NC = 2
NS = 16
NW = NC * NS
L = 16
RPW = E // NW
EPAD = E + L


def _sc_body(o_hbm, y_hbm, eb_hbm, part_hbm, o_v, y_v, eb_v, gacc_v, seg_s):
    wid = lax.axis_index("s") * NC + lax.axis_index("c")
    pltpu.sync_copy(o_hbm, o_v.at[pl.ds(0, E)])
    pltpu.sync_copy(y_hbm, y_v.at[pl.ds(0, E)])
    pltpu.sync_copy(eb_hbm, eb_v.at[pl.ds(0, E)])

    for g in range(NG):
        def bs_step(_, lohi, g=g):
            lo, hi = lohi
            mid = (lo + hi) // 2
            le = eb_v[pl.ds(mid, L)][0] <= g
            return jnp.where(le, mid + 1, lo), jnp.where(le, hi, mid)

        lo, _ = lax.fori_loop(
            0, 15, bs_step, (jnp.int32(0), jnp.int32(E))
        )
        seg_s[g] = lo

    lane = lax.iota(jnp.int32, L)
    zero16 = jnp.zeros((L,), jnp.float32)
    for g in range(NG):
        gacc_v[pl.ds(g * L, L)] = zero16

    def row_step(r, carry):
        i = wid + r * NW
        g = eb_v[pl.ds(i, L)][0]
        yi = y_v[pl.ds(i, L)][0]
        oi = o_v[pl.ds(i, L)][0]
        end = seg_s[g]
        kb0 = (i + 1) // L
        kb1 = (end + L - 1) // L
        kbl = jnp.maximum(kb1 - 1, kb0)
        has_last = kb1 - 1 > kb0

        def pair_loss(base):
            yv = y_v[pl.ds(base, L)]
            ov = o_v[pl.ds(base, L)]
            t = jnp.sign(yi - yv)
            return jnp.maximum(MARGIN - t * (oi - ov), 0.0)

        jv0 = kb0 * L + lane
        acc0 = jnp.where((jv0 > i) & (jv0 < end), pair_loss(kb0 * L), 0.0)
        jvl = kbl * L + lane
        hl = jnp.where(has_last, jnp.float32(1.0), jnp.float32(0.0))
        accl = jnp.where(jvl < end, pair_loss(kbl * L), 0.0) * hl

        def interior(kb, ai):
            return ai + pair_loss(kb * L)

        accin = plsc.parallel_loop(
            kb0 + 1, kbl, unroll=8, carry=acc0 + accl
        )(interior)

        goff = g * L
        gacc_v[pl.ds(goff, L)] = gacc_v[pl.ds(goff, L)] + accin
        return carry

    lax.fori_loop(0, RPW, row_step, jnp.int32(0))
    pltpu.sync_copy(gacc_v, part_hbm.at[wid])


def _sc_partials(outputs, y, edges_batch):
    mesh = plsc.VectorSubcoreMesh(
        core_axis_name="c", subcore_axis_name="s",
        num_cores=NC, num_subcores=NS,
    )
    f = pl.kernel(
        _sc_body,
        out_type=jax.ShapeDtypeStruct((NW, NG * L), jnp.float32),
        mesh=mesh,
        scratch_types=[
            pltpu.VMEM((EPAD,), jnp.float32),
            pltpu.VMEM((EPAD,), jnp.float32),
            pltpu.VMEM((EPAD,), jnp.int32),
            pltpu.VMEM((NG * L,), jnp.float32),
            pltpu.SMEM((NG,), jnp.int32),
        ],
    )
    return f(outputs, y, edges_batch)


def _finish_body(part_ref, eb_ref, out_ref):
    part = part_ref[...]
    eb = eb_ref[...]
    total = jnp.float32(0.0)
    for g in range(NG):
        n = jnp.sum((eb == g).astype(jnp.float32))
        cnt = n * (n - 1.0) * 0.5
        s = jnp.sum(part[:, g * L:(g + 1) * L])
        total = total + s / jnp.maximum(cnt, 1.0)
    num_graphs = jnp.max(eb).astype(jnp.float32) + 1.0
    out_ref[...] = (total / num_graphs).reshape(1, 1)


@jax.jit
def kernel(outputs, y, edges_batch):
    part = _sc_partials(outputs, y, edges_batch)
    eb2d = edges_batch.reshape(128, 128)
    out = pl.pallas_call(
        _finish_body,
        out_shape=jax.ShapeDtypeStruct((1, 1), jnp.float32),
    )(part, eb2d)
    return out[0, 0]


# speedup vs baseline: 1.3881x; 1.3881x over previous
import jax
import jax.numpy as jnp
from jax import lax
from jax.experimental import pallas as pl
from jax.experimental.pallas import tpu as pltpu
from jax.experimental.pallas import tpu_sc as plsc

E = 16384
NG = 16
MARGIN = 0.1
NC = 2
NS = 16
NW = NC * NS
L = 16
EPAD = E + 2 * L
SIGN_BIT = -2147483648


def _sc_body(o_hbm, y_hbm, eb_hbm, part_hbm, o_v, y_v, eb_v, gacc_v):
    wid = lax.axis_index("s") * NC + lax.axis_index("c")
    pltpu.sync_copy(o_hbm, o_v.at[pl.ds(0, E)])
    pltpu.sync_copy(y_hbm, y_v.at[pl.ds(0, E)])
    pltpu.sync_copy(eb_hbm, eb_v.at[pl.ds(0, E)])

    ends = []
    for g in range(NG):
        def bs_step(_, lohi, g=g):
            lo, hi = lohi
            mid = (lo + hi) // 2
            le = eb_v[pl.ds(mid, L)][0] <= g
            return jnp.where(le, mid + 1, lo), jnp.where(le, hi, mid)

        lo, _ = lax.fori_loop(0, 15, bs_step, (jnp.int32(0), jnp.int32(E)))
        ends.append(lo)

    lane = lax.iota(jnp.int32, L)
    zero16 = jnp.zeros((L,), jnp.float32)

    for g in range(NG):
        start = jnp.int32(0) if g == 0 else ends[g - 1]
        end = ends[g]
        r0 = (start - wid + NW - 1) // NW
        r1 = (end - wid + NW - 1) // NW

        def row_step(r, acc, end=end):
            i = wid + r * NW
            yi = y_v[pl.ds(i, L)][0]
            oi = o_v[pl.ds(i, L)][0]

            def pair_loss(base):
                yv = y_v[pl.ds(base, L)]
                ov = o_v[pl.ds(base, L)]
                dy = yi - yv
                do = oi - ov
                tdo = jnp.where(
                    dy > 0.0, do, jnp.where(dy < 0.0, 0.0 - do, 0.0)
                )
                return jnp.maximum(MARGIN - tdo, 0.0)

            kb0 = (i + 1) // L
            kb1 = (end + L - 1) // L
            kbl = jnp.maximum(kb1 - 1, kb0)

            jv0 = kb0 * L + lane
            a = jnp.where((jv0 > i) & (jv0 < end), pair_loss(kb0 * L), 0.0)
            jvl = kbl * L + lane
            hl = jnp.where(kb1 - 1 > kb0, jnp.float32(1.0), jnp.float32(0.0))
            a = a + jnp.where(jvl < end, pair_loss(kbl * L), 0.0) * hl

            lo_i = kb0 + 1
            n2 = jnp.maximum(kbl - lo_i, 0) >> 1
            up2 = lo_i + n2 * 2

            def interior(kb, accs):
                a0, a1 = accs
                return (a0 + pair_loss(kb * L), a1 + pair_loss(kb * L + L))

            a0, a1 = plsc.parallel_loop(
                lo_i, up2, step=2, unroll=4, carry=(a, zero16)
            )(interior)
            hodd = jnp.where(up2 < kbl, jnp.float32(1.0), jnp.float32(0.0))
            aodd = pair_loss(up2 * L) * hodd
            return acc + a0 + a1 + aodd

        acc_g = lax.fori_loop(r0, r1, row_step, zero16)
        gacc_v[pl.ds(g * L, L)] = acc_g

    pltpu.sync_copy(gacc_v, part_hbm.at[wid])


def _sc_partials(outputs, y, edges_batch):
    mesh = plsc.VectorSubcoreMesh(
        core_axis_name="c", subcore_axis_name="s",
        num_cores=NC, num_subcores=NS,
    )
    f = pl.kernel(
        _sc_body,
        out_type=jax.ShapeDtypeStruct((NW, NG * L), jnp.float32),
        mesh=mesh,
        scratch_types=[
            pltpu.VMEM((EPAD,), jnp.float32),
            pltpu.VMEM((EPAD,), jnp.float32),
            pltpu.VMEM((EPAD,), jnp.int32),
            pltpu.VMEM((NG * L,), jnp.float32),
        ],
    )
    return f(outputs, y, edges_batch)


def _finish_body(part_ref, eb_ref, out_ref):
    part = part_ref[...]
    eb = eb_ref[...]
    total = jnp.float32(0.0)
    for g in range(NG):
        n = jnp.sum((eb == g).astype(jnp.float32))
        cnt = n * (n - 1.0) * 0.5
        s = jnp.sum(part[:, g * L:(g + 1) * L])
        total = total + s / jnp.maximum(cnt, 1.0)
    num_graphs = jnp.max(eb).astype(jnp.float32) + 1.0
    out_ref[...] = (total / num_graphs).reshape(1, 1)


@jax.jit
def kernel(outputs, y, edges_batch):
    part = _sc_partials(outputs, y, edges_batch)
    eb2d = edges_batch.reshape(128, 128)
    out = pl.pallas_call(
        _finish_body,
        out_shape=jax.ShapeDtypeStruct((1, 1), jnp.float32),
    )(part, eb2d)
    return out[0, 0]
